# baseline (speedup 1.0000x reference)
import jax
import jax.numpy as jnp
from jax.experimental import pallas as pl
from jax.experimental.pallas import tpu as pltpu

_E = 64
_K = 8
_NS = 4
_TC = 256
_TB = _NS * _TC


def _gate_one_chunk(x, w):
    logits = jax.lax.dot_general(
        w, x, (((1,), (1,)), ((), ())),
        preferred_element_type=jnp.float32,
        precision=jax.lax.Precision.DEFAULT)
    mx = jnp.max(logits, axis=0, keepdims=True)
    ex = jnp.exp(logits - mx)
    probs = ex / jnp.sum(ex, axis=0, keepdims=True)

    row = jax.lax.broadcasted_iota(jnp.int32, probs.shape, 0)
    remaining = probs
    comb = jnp.zeros_like(probs)
    disp = jnp.zeros_like(probs)
    for _ in range(_K):
        m = jnp.max(remaining, axis=0, keepdims=True)
        cand = remaining == m
        sel_row = jnp.min(jnp.where(cand, row, _E), axis=0, keepdims=True)
        sel = row == sel_row
        comb = jnp.where(sel, probs, comb)
        disp = jnp.where(sel, 1.0, disp)
        remaining = jnp.where(sel, -1.0, remaining)

    re = jax.lax.broadcasted_iota(jnp.int32, (_E, _E), 0)
    ce = jax.lax.broadcasted_iota(jnp.int32, (_E, _E), 1)
    ident = (ce == re).astype(jnp.float32)
    dims = (((0,), (0,)), ((), ()))
    disp_t = jax.lax.dot_general(
        disp, ident, dims, preferred_element_type=jnp.float32,
        precision=jax.lax.Precision.DEFAULT)
    comb_t = jax.lax.dot_general(
        comb, ident, dims, preferred_element_type=jnp.float32,
        precision=jax.lax.Precision.HIGHEST)
    return disp_t, comb_t


def _moe_gate_kernel(x0_ref, x1_ref, x2_ref, x3_ref, w_ref,
                     disp_ref, comb_ref):
    w = w_ref[...]
    for c, x_ref in enumerate((x0_ref, x1_ref, x2_ref, x3_ref)):
        disp_t, comb_t = _gate_one_chunk(x_ref[0], w)
        disp_ref[0, c * _TC:(c + 1) * _TC, :] = disp_t
        comb_ref[0, c * _TC:(c + 1) * _TC, :] = comb_t


def kernel(hidden_states, W):
    b, s, d = hidden_states.shape
    e, _ = W.shape
    nsb = s // _TB
    ncb = s // _TC

    def x_spec(c):
        def imap(i, c=c):
            r = i * _NS + c
            return (r // ncb, r % ncb, 0)
        return pl.BlockSpec((1, _TC, d), imap)

    disp, comb = pl.pallas_call(
        _moe_gate_kernel,
        grid=(b * s // _TB,),
        in_specs=[x_spec(0), x_spec(1), x_spec(2), x_spec(3),
                  pl.BlockSpec((e, d), lambda i: (0, 0))],
        out_specs=[
            pl.BlockSpec((1, _TB, e), lambda i: (i // nsb, i % nsb, 0)),
            pl.BlockSpec((1, _TB, e), lambda i: (i // nsb, i % nsb, 0)),
        ],
        out_shape=[
            jax.ShapeDtypeStruct((b, s, e), jnp.float32),
            jax.ShapeDtypeStruct((b, s, e), jnp.float32),
        ],
        compiler_params=pltpu.CompilerParams(
            dimension_semantics=("parallel",)),
    )(hidden_states, hidden_states, hidden_states, hidden_states, W)
    zeros_rest = jnp.zeros((b, s, e, _K - 1), jnp.float32)
    dispatch_tensor = jnp.concatenate([disp[..., None], zeros_rest], axis=-1)
    combine_tensor = jnp.concatenate([comb[..., None], zeros_rest], axis=-1)
    return (dispatch_tensor, combine_tensor)


# speedup vs baseline: 1.1788x; 1.1788x over previous
import jax
import jax.numpy as jnp
from jax.experimental import pallas as pl
from jax.experimental.pallas import tpu as pltpu

_E = 64
_K = 8
_TB = 1024


def _moe_gate_kernel(x_ref, w_ref, disp_ref, comb_ref):
    x = x_ref[0]
    w = w_ref[...]
    logits = jax.lax.dot_general(
        w, x, (((1,), (1,)), ((), ())),
        preferred_element_type=jnp.float32,
        precision=jax.lax.Precision.DEFAULT)
    mx = jnp.max(logits, axis=0, keepdims=True)
    ex = jnp.exp(logits - mx)
    probs = ex / jnp.sum(ex, axis=0, keepdims=True)

    row = jax.lax.broadcasted_iota(jnp.int32, probs.shape, 0)
    remaining = probs
    comb = jnp.zeros_like(probs)
    disp = jnp.zeros_like(probs)
    for _ in range(_K):
        m = jnp.max(remaining, axis=0, keepdims=True)
        cand = remaining == m
        sel_row = jnp.min(jnp.where(cand, row, _E), axis=0, keepdims=True)
        sel = row == sel_row
        comb = jnp.where(sel, probs, comb)
        disp = jnp.where(sel, 1.0, disp)
        remaining = jnp.where(sel, -1.0, remaining)

    re = jax.lax.broadcasted_iota(jnp.int32, (_E, _E), 0)
    ce = jax.lax.broadcasted_iota(jnp.int32, (_E, _E), 1)
    ident = (ce == re).astype(jnp.float32)
    dims = (((0,), (0,)), ((), ()))
    disp_ref[...] = jax.lax.dot_general(
        disp, ident, dims, preferred_element_type=jnp.float32,
        precision=jax.lax.Precision.DEFAULT)[None]
    comb_ref[...] = jax.lax.dot_general(
        comb, ident, dims, preferred_element_type=jnp.float32,
        precision=jax.lax.Precision.HIGHEST)[None]


def kernel(hidden_states, W):
    b, s, d = hidden_states.shape
    e, _ = W.shape
    nsb = s // _TB
    disp, comb = pl.pallas_call(
        _moe_gate_kernel,
        grid=(b * nsb,),
        in_specs=[
            pl.BlockSpec((1, _TB, d), lambda i: (i // nsb, i % nsb, 0)),
            pl.BlockSpec((e, d), lambda i: (0, 0)),
        ],
        out_specs=[
            pl.BlockSpec((1, _TB, e), lambda i: (i // nsb, i % nsb, 0)),
            pl.BlockSpec((1, _TB, e), lambda i: (i // nsb, i % nsb, 0)),
        ],
        out_shape=[
            jax.ShapeDtypeStruct((b, s, e), jnp.float32),
            jax.ShapeDtypeStruct((b, s, e), jnp.float32),
        ],
        compiler_params=pltpu.CompilerParams(
            dimension_semantics=("arbitrary",)),
    )(hidden_states, W)
    zeros_rest = jnp.zeros((b, s, e, _K - 1), jnp.float32)
    dispatch_tensor = jnp.concatenate([disp[..., None], zeros_rest], axis=-1)
    combine_tensor = jnp.concatenate([comb[..., None], zeros_rest], axis=-1)
    return (dispatch_tensor, combine_tensor)


# speedup vs baseline: 1.1811x; 1.0020x over previous
import jax
import jax.numpy as jnp
from jax.experimental import pallas as pl

_E = 64
_K = 8
_TB = 1024


def _moe_gate_kernel(x_ref, w_ref, disp_ref, comb_ref):
    x = x_ref[0]
    w = w_ref[...]
    logits = jax.lax.dot_general(
        w, x, (((1,), (1,)), ((), ())),
        preferred_element_type=jnp.float32,
        precision=jax.lax.Precision.DEFAULT)
    mx = jnp.max(logits, axis=0, keepdims=True)
    ex = jnp.exp(logits - mx)
    probs = ex / jnp.sum(ex, axis=0, keepdims=True)

    row = jax.lax.broadcasted_iota(jnp.int32, probs.shape, 0)
    remaining = probs
    comb = jnp.zeros_like(probs)
    disp = jnp.zeros_like(probs)
    for _ in range(_K):
        m = jnp.max(remaining, axis=0, keepdims=True)
        cand = remaining == m
        sel_row = jnp.min(jnp.where(cand, row, _E), axis=0, keepdims=True)
        sel = row == sel_row
        comb = jnp.where(sel, probs, comb)
        disp = jnp.where(sel, 1.0, disp)
        remaining = jnp.where(sel, -1.0, remaining)

    re = jax.lax.broadcasted_iota(jnp.int32, (_E, _E), 0)
    ce = jax.lax.broadcasted_iota(jnp.int32, (_E, _E), 1)
    ident = (ce == re).astype(jnp.float32)
    dims = (((0,), (0,)), ((), ()))
    disp_ref[...] = jax.lax.dot_general(
        disp, ident, dims, preferred_element_type=jnp.float32,
        precision=jax.lax.Precision.DEFAULT)[None]
    comb_ref[...] = jax.lax.dot_general(
        comb, ident, dims, preferred_element_type=jnp.float32,
        precision=jax.lax.Precision.HIGHEST)[None]


def kernel(hidden_states, W):
    b, s, d = hidden_states.shape
    e, _ = W.shape
    nsb = s // _TB
    disp, comb = pl.pallas_call(
        _moe_gate_kernel,
        grid=(b * nsb,),
        in_specs=[
            pl.BlockSpec((1, _TB, d), lambda i: (i // nsb, i % nsb, 0)),
            pl.BlockSpec((e, d), lambda i: (0, 0)),
        ],
        out_specs=[
            pl.BlockSpec((1, _TB, e), lambda i: (i // nsb, i % nsb, 0)),
            pl.BlockSpec((1, _TB, e), lambda i: (i // nsb, i % nsb, 0)),
        ],
        out_shape=[
            jax.ShapeDtypeStruct((b, s, e), jnp.float32),
            jax.ShapeDtypeStruct((b, s, e), jnp.float32),
        ],
    )(hidden_states, W)
    zeros_rest = jnp.zeros((b, s, e, _K - 1), jnp.float32)
    dispatch_tensor = jnp.concatenate([disp[..., None], zeros_rest], axis=-1)
    combine_tensor = jnp.concatenate([comb[..., None], zeros_rest], axis=-1)
    return (dispatch_tensor, combine_tensor)
